# baseline (speedup 1.0000x reference)
import functools

import jax
import jax.numpy as jnp
from jax.experimental import pallas as pl
from jax.experimental.pallas import tpu as pltpu

N_NODES = 10
D_FEAT = 131

TN = 512
G1 = 4
G2 = 5
G3 = 10
G4 = 5
P1, P2, P3 = G1, G1 + G2, G1 + G2 + G3
STEPS = G1 + G2 + G3 + G4


def _rdot(x, wt):
    return jax.lax.dot_general(
        x, wt, (((1,), (0,)), ((), ())), preferred_element_type=jnp.float32)


def _mesh1_kernel(sp_ref, st_ref, gb_ref, w1t_ref, w2t_ref, w3t_ref, w4_ref,
                  b1_ref, b2_ref, b3_ref, b4_ref,
                  out1_ref, out2_ref, h1, h2c, f):
    s = pl.program_id(0)

    @pl.when(s == 0)
    def _gather():
        iota = jax.lax.broadcasted_iota(
            jnp.int32, (16, 16), 1).astype(jnp.float32)
        acc = jnp.zeros((16, 16), jnp.float32)
        for t in range(4):
            acc = acc + (gb_ref[:, D_FEAT + t:D_FEAT + t + 1] == iota
                         ).astype(jnp.float32)
        f2d = jax.lax.dot_general(
            acc, gb_ref[:, :D_FEAT], (((1,), (0,)), ((), ())),
            preferred_element_type=jnp.float32) * 0.25
        for i in range(N_NODES):
            f[:, D_FEAT * i:D_FEAT * (i + 1)] = f2d[i:i + 1, :]

    @pl.when(s < P1)
    def _phase1():
        h1[:, pl.ds(s * TN, TN)] = jax.nn.relu(
            _rdot(sp_ref[...], w1t_ref[:640, :])
            + _rdot(st_ref[...], w1t_ref[640:, :]) + b1_ref[...])

    @pl.when((s >= P1) & (s < P2))
    def _phase2():
        out1_ref[...] = _rdot(h1[:, :2000], w2t_ref[...]) + b2_ref[...]

    @pl.when((s >= P2) & (s < P3))
    def _phase3():
        j = s - P2
        y = jax.nn.relu(_rdot(f[...], w3t_ref[...]) + b3_ref[...])
        h2c[pl.ds(j * TN, TN), :] = jax.lax.transpose(y, (1, 0))

    @pl.when(s >= P3)
    def _phase4():
        y = jax.lax.dot_general(
            w4_ref[...], h2c[...], (((1,), (0,)), ((), ())),
            preferred_element_type=jnp.float32)
        out2_ref[...] = jax.lax.transpose(y, (1, 0)) + b4_ref[...]


@functools.partial(jax.jit, static_argnames=("interpret",))
def _run(spatial, structural, neighbour, W1, b1, W2, b2, W3, b3, W4, b4,
         interpret=False):
    smat = structural.reshape(N_NODES, D_FEAT)
    idxf = jnp.concatenate(
        [jnp.arange(N_NODES, dtype=jnp.float32)[:, None],
         neighbour.reshape(N_NODES, 3).astype(jnp.float32)], axis=1)
    gb = jnp.pad(jnp.concatenate([smat, idxf], axis=1), ((0, 6), (0, 9)))

    const = lambda bs: pl.BlockSpec(bs, lambda s: (0, 0))
    out1, out2 = pl.pallas_call(
        _mesh1_kernel,
        grid=(STEPS,),
        in_specs=[
            const((1, 640)),
            const((1, 1310)),
            const((16, 144)),
            pl.BlockSpec((1950, TN), lambda s: (0, jnp.minimum(s, G1 - 1))),
            pl.BlockSpec((2000, TN), lambda s: (0, jnp.clip(s - P1, 0, G2 - 1))),
            pl.BlockSpec((1310, TN), lambda s: (0, jnp.clip(s - P2, 0, G3 - 1))),
            pl.BlockSpec((TN, 5120), lambda s: (jnp.clip(s - P3, 0, G4 - 1), 0)),
            pl.BlockSpec((1, TN), lambda s: (0, jnp.minimum(s, G1 - 1))),
            pl.BlockSpec((1, TN), lambda s: (0, jnp.clip(s - P1, 0, G2 - 1))),
            pl.BlockSpec((1, TN), lambda s: (0, jnp.clip(s - P2, 0, G3 - 1))),
            pl.BlockSpec((1, TN), lambda s: (0, jnp.clip(s - P3, 0, G4 - 1))),
        ],
        out_specs=[
            pl.BlockSpec((1, TN), lambda s: (0, jnp.clip(s - P1, 0, G2 - 1))),
            pl.BlockSpec((1, TN), lambda s: (0, jnp.clip(s - P3, 0, G4 - 1))),
        ],
        out_shape=[
            jax.ShapeDtypeStruct((1, 2560), jnp.float32),
            jax.ShapeDtypeStruct((1, 2560), jnp.float32),
        ],
        scratch_shapes=[
            pltpu.VMEM((1, TN * G1), jnp.float32),
            pltpu.VMEM((5120, 1), jnp.float32),
            pltpu.VMEM((1, N_NODES * D_FEAT), jnp.float32),
        ],
        compiler_params=pltpu.CompilerParams(
            vmem_limit_bytes=56 * 1024 * 1024),
        interpret=interpret,
    )(spatial[None, :], structural[None, :], gb, W1.T, W2.T, W3.T, W4,
      b1[None, :], b2[None, :], b3[None, :], b4[None, :])
    return out1.reshape(2560), out2.reshape(2560)


def kernel(spatial, structural, neighbour, W1, b1, W2, b2, W3, b3, W4, b4):
    return _run(spatial, structural, neighbour,
                W1, b1, W2, b2, W3, b3, W4, b4)


# speedup vs baseline: 1.0594x; 1.0594x over previous
import functools

import jax
import jax.numpy as jnp
from jax.experimental import pallas as pl
from jax.experimental.pallas import tpu as pltpu

N_NODES = 10
D_FEAT = 131

T1, G1 = 1024, 2
T2, G2 = 640, 4
T3, G3 = 1024, 5
T4, G4 = 512, 5
P1, P2, P3 = G1, G1 + G2, G1 + G2 + G3
STEPS = G1 + G2 + G3 + G4


def _rdot(x, wt):
    return jax.lax.dot_general(
        x, wt, (((1,), (0,)), ((), ())), preferred_element_type=jnp.float32)


def _mesh1_kernel(sp_ref, st_ref, gb_ref, w1t_ref, w2t_ref, w3t_ref, w4_ref,
                  b1_ref, b2_ref, b3_ref, b4_ref,
                  out1_ref, out2_ref, h1, h2c, f):
    s = pl.program_id(0)

    @pl.when(s == 0)
    def _gather():
        iota = jax.lax.broadcasted_iota(
            jnp.int32, (16, 16), 1).astype(jnp.float32)
        acc = jnp.zeros((16, 16), jnp.float32)
        for t in range(4):
            acc = acc + (gb_ref[:, D_FEAT + t:D_FEAT + t + 1] == iota
                         ).astype(jnp.float32)
        f2d = jax.lax.dot_general(
            acc, gb_ref[:, :D_FEAT], (((1,), (0,)), ((), ())),
            preferred_element_type=jnp.float32) * 0.25
        for i in range(N_NODES):
            f[:, D_FEAT * i:D_FEAT * (i + 1)] = f2d[i:i + 1, :]

    @pl.when(s < P1)
    def _phase1():
        h1[:, pl.ds(s * T1, T1)] = jax.nn.relu(
            _rdot(sp_ref[...], w1t_ref[:640, :])
            + _rdot(st_ref[...], w1t_ref[640:, :]) + b1_ref[...])

    @pl.when((s >= P1) & (s < P2))
    def _phase2():
        out1_ref[...] = _rdot(h1[:, :2000], w2t_ref[...]) + b2_ref[...]

    @pl.when((s >= P2) & (s < P3))
    def _phase3():
        j = s - P2
        y = jax.nn.relu(_rdot(f[...], w3t_ref[...]) + b3_ref[...])
        h2c[pl.ds(j * T3, T3), :] = jax.lax.transpose(y, (1, 0))

    @pl.when(s >= P3)
    def _phase4():
        y = jax.lax.dot_general(
            w4_ref[...], h2c[...], (((1,), (0,)), ((), ())),
            preferred_element_type=jnp.float32)
        out2_ref[...] = jax.lax.transpose(y, (1, 0)) + b4_ref[...]


@functools.partial(jax.jit, static_argnames=("interpret",))
def _run(spatial, structural, neighbour, W1, b1, W2, b2, W3, b3, W4, b4,
         interpret=False):
    smat = structural.reshape(N_NODES, D_FEAT)
    idxf = jnp.concatenate(
        [jnp.arange(N_NODES, dtype=jnp.float32)[:, None],
         neighbour.reshape(N_NODES, 3).astype(jnp.float32)], axis=1)
    gb = jnp.pad(jnp.concatenate([smat, idxf], axis=1), ((0, 6), (0, 9)))

    const = lambda bs: pl.BlockSpec(bs, lambda s: (0, 0))
    out1, out2 = pl.pallas_call(
        _mesh1_kernel,
        grid=(STEPS,),
        in_specs=[
            const((1, 640)),
            const((1, 1310)),
            const((16, 144)),
            pl.BlockSpec((1950, T1), lambda s: (0, jnp.minimum(s, G1 - 1))),
            pl.BlockSpec((2000, T2), lambda s: (0, jnp.clip(s - P1, 0, G2 - 1))),
            pl.BlockSpec((1310, T3), lambda s: (0, jnp.clip(s - P2, 0, G3 - 1))),
            pl.BlockSpec((T4, 5120), lambda s: (jnp.clip(s - P3, 0, G4 - 1), 0)),
            pl.BlockSpec((1, T1), lambda s: (0, jnp.minimum(s, G1 - 1))),
            pl.BlockSpec((1, T2), lambda s: (0, jnp.clip(s - P1, 0, G2 - 1))),
            pl.BlockSpec((1, T3), lambda s: (0, jnp.clip(s - P2, 0, G3 - 1))),
            pl.BlockSpec((1, T4), lambda s: (0, jnp.clip(s - P3, 0, G4 - 1))),
        ],
        out_specs=[
            pl.BlockSpec((1, T2), lambda s: (0, jnp.clip(s - P1, 0, G2 - 1))),
            pl.BlockSpec((1, T4), lambda s: (0, jnp.clip(s - P3, 0, G4 - 1))),
        ],
        out_shape=[
            jax.ShapeDtypeStruct((1, 2560), jnp.float32),
            jax.ShapeDtypeStruct((1, 2560), jnp.float32),
        ],
        scratch_shapes=[
            pltpu.VMEM((1, T1 * G1), jnp.float32),
            pltpu.VMEM((5120, 1), jnp.float32),
            pltpu.VMEM((1, N_NODES * D_FEAT), jnp.float32),
        ],
        compiler_params=pltpu.CompilerParams(
            vmem_limit_bytes=64 * 1024 * 1024),
        interpret=interpret,
    )(spatial[None, :], structural[None, :], gb, W1.T, W2.T, W3.T, W4,
      b1[None, :], b2[None, :], b3[None, :], b4[None, :])
    return out1.reshape(2560), out2.reshape(2560)


def kernel(spatial, structural, neighbour, W1, b1, W2, b2, W3, b3, W4, b4):
    return _run(spatial, structural, neighbour,
                W1, b1, W2, b2, W3, b3, W4, b4)


# speedup vs baseline: 1.0657x; 1.0060x over previous
import functools

import jax
import jax.numpy as jnp
from jax.experimental import pallas as pl
from jax.experimental.pallas import tpu as pltpu

N_NODES = 10
D_FEAT = 131

T1, G1 = 1024, 2
T2, G2 = 640, 4
T3, G3 = 1280, 4
T4, G4 = 512, 5
P1, P2, P3 = G1, G1 + G2, G1 + G2 + G3
STEPS = G1 + G2 + G3 + G4


def _rdot(x, wt):
    return jax.lax.dot_general(
        x, wt, (((1,), (0,)), ((), ())), preferred_element_type=jnp.float32)


def _mesh1_kernel(sp_ref, st_ref, gb_ref, w1t_ref, w2t_ref, w3t_ref, w4_ref,
                  b1_ref, b2_ref, b3_ref, b4_ref,
                  out1_ref, out2_ref, h1, h2c, f):
    s = pl.program_id(0)

    @pl.when(s == 0)
    def _gather():
        iota = jax.lax.broadcasted_iota(
            jnp.int32, (16, 16), 1).astype(jnp.float32)
        acc = jnp.zeros((16, 16), jnp.float32)
        for t in range(4):
            acc = acc + (gb_ref[:, D_FEAT + t:D_FEAT + t + 1] == iota
                         ).astype(jnp.float32)
        f2d = jax.lax.dot_general(
            acc, gb_ref[:, :D_FEAT], (((1,), (0,)), ((), ())),
            preferred_element_type=jnp.float32) * 0.25
        for i in range(N_NODES):
            f[:, D_FEAT * i:D_FEAT * (i + 1)] = f2d[i:i + 1, :]

    @pl.when(s < P1)
    def _phase1():
        h1[:, pl.ds(s * T1, T1)] = jax.nn.relu(
            _rdot(sp_ref[...], w1t_ref[:640, :])
            + _rdot(st_ref[...], w1t_ref[640:, :]) + b1_ref[...])

    @pl.when((s >= P1) & (s < P2))
    def _phase2():
        out1_ref[...] = _rdot(h1[:, :2000], w2t_ref[...]) + b2_ref[...]

    @pl.when((s >= P2) & (s < P3))
    def _phase3():
        j = s - P2
        y = jax.nn.relu(_rdot(f[...], w3t_ref[...]) + b3_ref[...])
        h2c[pl.ds(j * T3, T3), :] = jax.lax.transpose(y, (1, 0))

    @pl.when(s >= P3)
    def _phase4():
        y = jax.lax.dot_general(
            w4_ref[...], h2c[...], (((1,), (0,)), ((), ())),
            preferred_element_type=jnp.float32)
        out2_ref[...] = jax.lax.transpose(y, (1, 0)) + b4_ref[...]


@functools.partial(jax.jit, static_argnames=("interpret",))
def _run(spatial, structural, neighbour, W1, b1, W2, b2, W3, b3, W4, b4,
         interpret=False):
    smat = structural.reshape(N_NODES, D_FEAT)
    idxf = jnp.concatenate(
        [jnp.arange(N_NODES, dtype=jnp.float32)[:, None],
         neighbour.reshape(N_NODES, 3).astype(jnp.float32)], axis=1)
    gb = jnp.pad(jnp.concatenate([smat, idxf], axis=1), ((0, 6), (0, 9)))

    const = lambda bs: pl.BlockSpec(bs, lambda s: (0, 0))
    out1, out2 = pl.pallas_call(
        _mesh1_kernel,
        grid=(STEPS,),
        in_specs=[
            const((1, 640)),
            const((1, 1310)),
            const((16, 144)),
            pl.BlockSpec((1950, T1), lambda s: (0, jnp.minimum(s, G1 - 1))),
            pl.BlockSpec((2000, T2), lambda s: (0, jnp.clip(s - P1, 0, G2 - 1))),
            pl.BlockSpec((1310, T3), lambda s: (0, jnp.clip(s - P2, 0, G3 - 1))),
            pl.BlockSpec((T4, 5120), lambda s: (jnp.clip(s - P3, 0, G4 - 1), 0)),
            pl.BlockSpec((1, T1), lambda s: (0, jnp.minimum(s, G1 - 1))),
            pl.BlockSpec((1, T2), lambda s: (0, jnp.clip(s - P1, 0, G2 - 1))),
            pl.BlockSpec((1, T3), lambda s: (0, jnp.clip(s - P2, 0, G3 - 1))),
            pl.BlockSpec((1, T4), lambda s: (0, jnp.clip(s - P3, 0, G4 - 1))),
        ],
        out_specs=[
            pl.BlockSpec((1, T2), lambda s: (0, jnp.clip(s - P1, 0, G2 - 1))),
            pl.BlockSpec((1, T4), lambda s: (0, jnp.clip(s - P3, 0, G4 - 1))),
        ],
        out_shape=[
            jax.ShapeDtypeStruct((1, 2560), jnp.float32),
            jax.ShapeDtypeStruct((1, 2560), jnp.float32),
        ],
        scratch_shapes=[
            pltpu.VMEM((1, T1 * G1), jnp.float32),
            pltpu.VMEM((5120, 1), jnp.float32),
            pltpu.VMEM((1, N_NODES * D_FEAT), jnp.float32),
        ],
        compiler_params=pltpu.CompilerParams(
            vmem_limit_bytes=64 * 1024 * 1024),
        interpret=interpret,
    )(spatial[None, :], structural[None, :], gb, W1.T, W2.T, W3.T, W4,
      b1[None, :], b2[None, :], b3[None, :], b4[None, :])
    return out1.reshape(2560), out2.reshape(2560)


def kernel(spatial, structural, neighbour, W1, b1, W2, b2, W3, b3, W4, b4):
    return _run(spatial, structural, neighbour,
                W1, b1, W2, b2, W3, b3, W4, b4)
